# baseline (speedup 1.0000x reference)
import functools

import jax
import jax.numpy as jnp
from jax import lax
from jax.experimental import pallas as pl
from jax.experimental.pallas import tpu as pltpu
from jax.experimental.pallas import tpu_sc as plsc

NLAT, NLON = 721, 1440
C = 32
ROWS = C * NLAT
R = 16
NCHUNK = ROWS // R
NTEC = 32
TPT = -(-NCHUNK // NTEC)
L = 16
KCOL = NLON // L


def _sc_lerp(data2, ga, gb, wg):
    mesh = plsc.VectorSubcoreMesh(core_axis_name="c", subcore_axis_name="s")

    @functools.partial(
        pl.kernel,
        out_type=jax.ShapeDtypeStruct((ROWS, NLON), jnp.float32),
        mesh=mesh,
        compiler_params=pltpu.CompilerParams(use_tc_tiling_on_sc=False),
        scratch_types=[
            pltpu.VMEM((R,), jnp.int32),
            pltpu.VMEM((R,), jnp.int32),
            pltpu.VMEM((R, L), jnp.float32),
            pltpu.VMEM((R, NLON), jnp.float32),
            pltpu.VMEM((R, NLON), jnp.float32),
            pltpu.VMEM((R, NLON), jnp.float32),
            pltpu.SemaphoreType.DMA,
            pltpu.SemaphoreType.DMA,
        ],
    )
    def k(data_hbm, ga_hbm, gb_hbm, wg_hbm, out_hbm,
          ia_v, ib_v, w_v, a_v, b_v, o_v, sem_a, sem_b):
        wid = lax.axis_index("s") * 2 + lax.axis_index("c")

        def chunk_body(t, carry):
            j = wid + NTEC * t

            @pl.when(j < NCHUNK)
            def _():
                base = j * R
                pltpu.sync_copy(ga_hbm.at[pl.ds(base, R)], ia_v)
                pltpu.sync_copy(gb_hbm.at[pl.ds(base, R)], ib_v)
                pltpu.sync_copy(wg_hbm.at[pl.ds(base, R)], w_v)
                cpa = pltpu.async_copy(data_hbm.at[ia_v], a_v, sem_a)
                cpb = pltpu.async_copy(data_hbm.at[ib_v], b_v, sem_b)
                cpa.wait()
                cpb.wait()

                def row_body(r, rc):
                    wv = w_v[r, pl.ds(0, L)]

                    def col_body(kk, kc):
                        c0 = kk * L
                        av = a_v[r, pl.ds(c0, L)]
                        bv = b_v[r, pl.ds(c0, L)]
                        o_v[r, pl.ds(c0, L)] = av + wv * (bv - av)
                        return kc

                    lax.fori_loop(0, KCOL, col_body, 0)
                    return rc

                lax.fori_loop(0, R, row_body, 0)
                pltpu.sync_copy(o_v, out_hbm.at[pl.ds(base, R)])

            return carry

        lax.fori_loop(0, TPT, chunk_body, 0)

    return k(data2, ga, gb, wg)


def kernel(data, indices, interp_weights):
    idx = indices.astype(jnp.int32)
    ga = (jnp.arange(C, dtype=jnp.int32)[:, None] * NLAT + idx[None, :]).reshape(-1)
    gb = ga + 1
    wcol = jnp.tile(interp_weights.reshape(NLAT).astype(jnp.float32), C)
    wg = jnp.broadcast_to(wcol[:, None], (ROWS, L))
    data2 = data.reshape(ROWS, NLON)
    out = _sc_lerp(data2, ga, gb, wg)
    return out.reshape(data.shape)


# speedup vs baseline: 1.1969x; 1.1969x over previous
import functools

import jax
import jax.numpy as jnp
from jax import lax
from jax.experimental import pallas as pl
from jax.experimental.pallas import tpu as pltpu
from jax.experimental.pallas import tpu_sc as plsc

NLAT, NLON = 721, 1440
C = 32
ROWS = C * NLAT
R = 8
NCHUNK = ROWS // R
NTEC = 32
BASE_CH = NCHUNK // NTEC
NPAIR = BASE_CH // 2
L = 16
UNROLL = 6
KSTEPS = NLON // (L * UNROLL)


def _lerp_chunk(ab_v, w_v, o_v):
    wvs = [w_v[r, pl.ds(0, L)] for r in range(R)]

    def col_body(k, carry):
        base = k * (L * UNROLL)
        for u in range(UNROLL):
            c0 = base + u * L
            for r in range(R):
                av = ab_v[r, pl.ds(c0, L)]
                bv = ab_v[R + r, pl.ds(c0, L)]
                o_v[r, pl.ds(c0, L)] = av + wvs[r] * (bv - av)
        return carry

    lax.fori_loop(0, KSTEPS, col_body, 0)


def _sc_lerp(data2, pk, wg):
    mesh = plsc.VectorSubcoreMesh(core_axis_name="c", subcore_axis_name="s")

    @functools.partial(
        pl.kernel,
        out_type=jax.ShapeDtypeStruct((ROWS, NLON), jnp.float32),
        mesh=mesh,
        compiler_params=pltpu.CompilerParams(use_tc_tiling_on_sc=False),
        scratch_types=[
            pltpu.VMEM((2 * R,), jnp.int32),
            pltpu.VMEM((2 * R,), jnp.int32),
            pltpu.VMEM((R, L), jnp.float32),
            pltpu.VMEM((R, L), jnp.float32),
            pltpu.VMEM((2 * R, NLON), jnp.float32),
            pltpu.VMEM((2 * R, NLON), jnp.float32),
            pltpu.VMEM((R, NLON), jnp.float32),
            pltpu.VMEM((R, NLON), jnp.float32),
            pltpu.SemaphoreType.DMA,
            pltpu.SemaphoreType.DMA,
            pltpu.SemaphoreType.DMA,
            pltpu.SemaphoreType.DMA,
            pltpu.SemaphoreType.DMA,
            pltpu.SemaphoreType.DMA,
        ],
    )
    def k(data_hbm, pk_hbm, wg_hbm, out_hbm,
          pk0, pk1, w0, w1, ab0, ab1, o0, o1,
          semP0, semP1, semG0, semG1, semO0, semO1):
        wid = lax.axis_index("s") * 2 + lax.axis_index("c")
        start = wid * BASE_CH

        def pack_issue(c, pk_v, w_v, sem):
            pltpu.async_copy(pk_hbm.at[c], pk_v, sem)
            pltpu.async_copy(wg_hbm.at[c], w_v, sem)

        def pack_wait(pk_v, w_v, sem):
            pltpu.make_async_copy(pk_hbm.at[0], pk_v, sem).wait()
            pltpu.make_async_copy(wg_hbm.at[0], w_v, sem).wait()

        def gather_issue(pk_v, ab_v, sem):
            pltpu.async_copy(data_hbm.at[pk_v], ab_v, sem)

        def gather_wait(pk_v, ab_v, sem):
            pltpu.make_async_copy(data_hbm.at[pk_v], ab_v, sem).wait()

        def out_issue(c, o_v, sem):
            pltpu.async_copy(o_v, out_hbm.at[pl.ds(c * R, R)], sem)

        def out_wait(o_v, sem):
            pltpu.make_async_copy(o_v, out_hbm.at[pl.ds(0, R)], sem).wait()

        pack_issue(start, pk0, w0, semP0)
        pack_issue(start + 1, pk1, w1, semP1)
        pack_wait(pk0, w0, semP0)
        gather_issue(pk0, ab0, semG0)

        def pair_body(m, carry):
            cA = start + 2 * m
            cB = cA + 1

            gather_wait(pk0, ab0, semG0)
            pack_wait(pk1, w1, semP1)
            gather_issue(pk1, ab1, semG1)

            @pl.when(m > 0)
            def _():
                out_wait(o0, semO0)

            _lerp_chunk(ab0, w0, o0)

            @pl.when(m < NPAIR - 1)
            def _():
                pack_issue(cA + 2, pk0, w0, semP0)

            out_issue(cA, o0, semO0)
            gather_wait(pk1, ab1, semG1)

            @pl.when(m > 0)
            def _():
                out_wait(o1, semO1)

            @pl.when(m < NPAIR - 1)
            def _():
                pack_wait(pk0, w0, semP0)
                gather_issue(pk0, ab0, semG0)

            _lerp_chunk(ab1, w1, o1)

            @pl.when(m < NPAIR - 1)
            def _():
                pack_issue(cB + 2, pk1, w1, semP1)

            out_issue(cB, o1, semO1)
            return carry

        lax.fori_loop(0, NPAIR, pair_body, 0)
        out_wait(o0, semO0)
        out_wait(o1, semO1)

        @pl.when(wid < NCHUNK - NTEC * BASE_CH)
        def _():
            ce = NTEC * BASE_CH + wid
            pack_issue(ce, pk0, w0, semP0)
            pack_wait(pk0, w0, semP0)
            gather_issue(pk0, ab0, semG0)
            gather_wait(pk0, ab0, semG0)
            _lerp_chunk(ab0, w0, o0)
            out_issue(ce, o0, semO0)
            out_wait(o0, semO0)

    return k(data2, pk, wg)


def kernel(data, indices, interp_weights):
    idx = indices.astype(jnp.int32)
    ga = (jnp.arange(C, dtype=jnp.int32)[:, None] * NLAT + idx[None, :]).reshape(-1)
    pk = jnp.concatenate(
        [ga.reshape(NCHUNK, R), ga.reshape(NCHUNK, R) + 1], axis=1)
    wcol = jnp.tile(interp_weights.reshape(NLAT).astype(jnp.float32), C)
    wg = jnp.broadcast_to(wcol.reshape(NCHUNK, R, 1), (NCHUNK, R, L))
    data2 = data.reshape(ROWS, NLON)
    out = _sc_lerp(data2, pk, wg)
    return out.reshape(data.shape)
